# baseline (speedup 1.0000x reference)
import jax
import jax.numpy as jnp
from jax import lax
from jax.experimental import pallas as pl
from jax.experimental.pallas import tpu as pltpu
from jax.experimental.pallas import tpu_sc as plsc

NC = 2
NS = 16
NW = NC * NS

NROW = 4096
SEQ = 50
D = 64
RB = NROW // NW
L = 16


def _emb_body(x_hbm, table_hbm, out_hbm, idx_v, list_v, g_v, t_v,
              gsem0, gsem1, wsem0, wsem1):
    wid = lax.axis_index("s") * NC + lax.axis_index("c")

    pltpu.sync_copy(x_hbm.at[pl.ds(wid * RB, RB)], idx_v)

    gsems = (gsem0, gsem1)
    wsems = (wsem0, wsem1)

    def build_list(s, b):
        for k in range(RB // L):
            rows = lax.iota(jnp.int32, L) + k * L
            vals = plsc.load_gather(idx_v, [rows, jnp.full((L,), s, jnp.int32)])
            list_v[b, pl.ds(k * L, L)] = vals

    def fire_gather(b):
        pltpu.async_copy(table_hbm.at[list_v.at[b]], g_v.at[b], gsems[b])

    dbase = lax.iota(jnp.int32, L)
    dts = [lax.shift_right_logical(dbase + j * L, 3) for j in range(D // L)]
    drs = [jnp.bitwise_and(dbase + j * L, 7) for j in range(D // L)]

    def do_s(s, b):
        @pl.when(s >= 2)
        def _():
            pltpu.make_async_copy(
                t_v.at[b, :, :, pl.ds(0, RB)],
                out_hbm.at[s - 2, :, wid],
                wsems[b],
            ).wait()

        pltpu.make_async_copy(
            table_hbm.at[list_v.at[b]], g_v.at[b], gsems[b]
        ).wait()

        def tr_row(i0):
            for u in range(4):
                i = i0 + u
                col = jnp.full((L,), i, jnp.int32)
                for j in range(D // L):
                    vals = g_v[b, i, pl.ds(j * L, L)]
                    plsc.store_scatter(t_v.at[b], [dts[j], drs[j], col], vals)

        pl.loop(0, RB, step=4)(tr_row)

        pltpu.async_copy(
            t_v.at[b, :, :, pl.ds(0, RB)],
            out_hbm.at[s, :, wid],
            wsems[b],
        )

        @pl.when(s + 2 < SEQ)
        def _():
            build_list(s + 2, b)
            fire_gather(b)

    for b in range(2):
        build_list(b, b)
        fire_gather(b)

    def grp(g):
        do_s(g, 0)
        do_s(g + 1, 1)

    pl.loop(0, SEQ, step=2)(grp)

    for b in range(2):
        s = SEQ - 2 + b
        pltpu.make_async_copy(
            t_v.at[b, :, :, pl.ds(0, RB)],
            out_hbm.at[s, :, wid],
            wsems[b],
        ).wait()


@jax.jit
def kernel(x, table):
    mesh = plsc.VectorSubcoreMesh(core_axis_name="c", subcore_axis_name="s")
    tmp = pl.kernel(
        _emb_body,
        out_type=jax.ShapeDtypeStruct((SEQ, D // 8, NW, 8, RB), jnp.float32),
        mesh=mesh,
        scratch_types=[
            pltpu.VMEM((RB, SEQ), jnp.int32),
            pltpu.VMEM((2, RB), jnp.int32),
            pltpu.VMEM((2, RB, D), jnp.float32),
            pltpu.VMEM((2, D // 8, 8, 137), jnp.float32),
            pltpu.SemaphoreType.DMA,
            pltpu.SemaphoreType.DMA,
            pltpu.SemaphoreType.DMA,
            pltpu.SemaphoreType.DMA,
        ],
        compiler_params=pltpu.CompilerParams(
            use_tc_tiling_on_sc=False, needs_layout_passes=False
        ),
    )(x, table)
    return tmp.transpose(2, 4, 0, 1, 3).reshape(NROW, SEQ, D)


# speedup vs baseline: 1.0479x; 1.0479x over previous
import jax
import jax.numpy as jnp
from jax import lax
from jax.experimental import pallas as pl
from jax.experimental.pallas import tpu as pltpu
from jax.experimental.pallas import tpu_sc as plsc

NC = 2
NS = 16
NW = NC * NS

NROW = 4096
SEQ = 50
D = 64
RB = NROW // NW
L = 16


def _emb_body(x_hbm, table_hbm, out_hbm, idx_v, list_v, g_v, t_v,
              gsem0, gsem1, wsem0, wsem1):
    wid = lax.axis_index("s") * NC + lax.axis_index("c")

    pltpu.sync_copy(x_hbm.at[pl.ds(wid * RB, RB)], idx_v)

    gsems = (gsem0, gsem1)
    wsems = (wsem0, wsem1)

    def build_list(s, b):
        for k in range(RB // L):
            rows = lax.iota(jnp.int32, L) + k * L
            vals = plsc.load_gather(idx_v, [rows, jnp.full((L,), s, jnp.int32)])
            list_v[b, pl.ds(k * L, L)] = vals

    def fire_gather(b):
        pltpu.async_copy(table_hbm.at[list_v.at[b]], g_v.at[b], gsems[b])

    dbase = lax.iota(jnp.int32, L)

    def do_s(s, b):
        @pl.when(s >= 2)
        def _():
            for dt in range(D // 8):
                pltpu.make_async_copy(
                    t_v.at[b, pl.ds(dt * 8, 8), pl.ds(0, RB)],
                    out_hbm.at[s - 2, dt, wid],
                    wsems[b],
                ).wait()

        pltpu.make_async_copy(
            table_hbm.at[list_v.at[b]], g_v.at[b], gsems[b]
        ).wait()

        def tr_row(i0):
            for u in range(4):
                i = i0 + u
                col = jnp.full((L,), i, jnp.int32)
                for j in range(D // L):
                    vals = g_v[b, i, pl.ds(j * L, L)]
                    plsc.store_scatter(t_v.at[b], [dbase + j * L, col], vals)

        pl.loop(0, RB, step=4)(tr_row)

        for dt in range(D // 8):
            pltpu.async_copy(
                t_v.at[b, pl.ds(dt * 8, 8), pl.ds(0, RB)],
                out_hbm.at[s, dt, wid],
                wsems[b],
            )

        @pl.when(s + 2 < SEQ)
        def _():
            build_list(s + 2, b)
            fire_gather(b)

    for b in range(2):
        build_list(b, b)
        fire_gather(b)

    def grp(g):
        do_s(g, 0)
        do_s(g + 1, 1)

    pl.loop(0, SEQ, step=2)(grp)

    for b in range(2):
        s = SEQ - 2 + b
        for dt in range(D // 8):
            pltpu.make_async_copy(
                t_v.at[b, pl.ds(dt * 8, 8), pl.ds(0, RB)],
                out_hbm.at[s, dt, wid],
                wsems[b],
            ).wait()


@jax.jit
def kernel(x, table):
    mesh = plsc.VectorSubcoreMesh(core_axis_name="c", subcore_axis_name="s")
    tmp = pl.kernel(
        _emb_body,
        out_type=jax.ShapeDtypeStruct((SEQ, D // 8, NW, 8, RB), jnp.float32),
        mesh=mesh,
        scratch_types=[
            pltpu.VMEM((RB, SEQ), jnp.int32),
            pltpu.VMEM((2, RB), jnp.int32),
            pltpu.VMEM((2, RB, D), jnp.float32),
            pltpu.VMEM((2, D, 129), jnp.float32),
            pltpu.SemaphoreType.DMA,
            pltpu.SemaphoreType.DMA,
            pltpu.SemaphoreType.DMA,
            pltpu.SemaphoreType.DMA,
        ],
        compiler_params=pltpu.CompilerParams(
            use_tc_tiling_on_sc=False, needs_layout_passes=False
        ),
    )(x, table)
    return tmp.transpose(2, 4, 0, 1, 3).reshape(NROW, SEQ, D)


# speedup vs baseline: 1.0491x; 1.0012x over previous
import jax
import jax.numpy as jnp
from jax import lax
from jax.experimental import pallas as pl
from jax.experimental.pallas import tpu as pltpu
from jax.experimental.pallas import tpu_sc as plsc

NC = 2
NS = 16
NW = NC * NS

NROW = 4096
SEQ = 50
D = 64
RB = NROW // NW
L = 16


def _emb_body(x_hbm, table_hbm, out_hbm, idx_v, list_v, g_v, t_v, drain_v,
              gsem0, gsem1, wsem0, wsem1):
    wid = lax.axis_index("s") * NC + lax.axis_index("c")

    pltpu.sync_copy(x_hbm.at[pl.ds(wid * RB, RB)], idx_v)

    gsems = (gsem0, gsem1)
    wsems = (wsem0, wsem1)

    def build_list(s, b):
        for k in range(RB // L):
            rows = lax.iota(jnp.int32, L) + k * L
            vals = plsc.load_gather(idx_v, [rows, jnp.full((L,), s, jnp.int32)])
            list_v[b, pl.ds(k * L, L)] = vals

    def fire_gather(b):
        pltpu.async_copy(table_hbm.at[list_v.at[b]], g_v.at[b], gsems[b])

    dbase = lax.iota(jnp.int32, L)

    def do_s(s, b):
        @pl.when(s >= 2)
        def _():
            pltpu.make_async_copy(
                out_hbm.at[s - 2, :, wid], drain_v, wsems[b]
            ).wait()

        pltpu.make_async_copy(
            table_hbm.at[list_v.at[b]], g_v.at[b], gsems[b]
        ).wait()

        def tr_row(i0):
            for u in range(4):
                i = i0 + u
                col = jnp.full((L,), i, jnp.int32)
                for j in range(D // L):
                    vals = g_v[b, i, pl.ds(j * L, L)]
                    plsc.store_scatter(t_v.at[b], [dbase + j * L, col], vals)

        pl.loop(0, RB, step=4)(tr_row)

        for dt in range(D // 8):
            pltpu.async_copy(
                t_v.at[b, pl.ds(dt * 8, 8), pl.ds(0, RB)],
                out_hbm.at[s, dt, wid],
                wsems[b],
            )

        @pl.when(s + 2 < SEQ)
        def _():
            build_list(s + 2, b)
            fire_gather(b)

    for b in range(2):
        build_list(b, b)
        fire_gather(b)

    def grp(g):
        do_s(g, 0)
        do_s(g + 1, 1)

    pl.loop(0, SEQ, step=2)(grp)

    for b in range(2):
        s = SEQ - 2 + b
        pltpu.make_async_copy(
            out_hbm.at[s, :, wid], drain_v, wsems[b]
        ).wait()


@jax.jit
def kernel(x, table):
    mesh = plsc.VectorSubcoreMesh(core_axis_name="c", subcore_axis_name="s")
    tmp = pl.kernel(
        _emb_body,
        out_type=jax.ShapeDtypeStruct((SEQ, D // 8, NW, 8, RB), jnp.float32),
        mesh=mesh,
        scratch_types=[
            pltpu.VMEM((RB, SEQ), jnp.int32),
            pltpu.VMEM((2, RB), jnp.int32),
            pltpu.VMEM((2, RB, D), jnp.float32),
            pltpu.VMEM((2, D, 129), jnp.float32),
            pltpu.VMEM((D // 8, 8, RB), jnp.float32),
            pltpu.SemaphoreType.DMA,
            pltpu.SemaphoreType.DMA,
            pltpu.SemaphoreType.DMA,
            pltpu.SemaphoreType.DMA,
        ],
        compiler_params=pltpu.CompilerParams(
            use_tc_tiling_on_sc=False, needs_layout_passes=False
        ),
    )(x, table)
    return tmp.transpose(2, 4, 0, 1, 3).reshape(NROW, SEQ, D)


# speedup vs baseline: 1.0509x; 1.0017x over previous
import jax
import jax.numpy as jnp
from jax import lax
from jax.experimental import pallas as pl
from jax.experimental.pallas import tpu as pltpu
from jax.experimental.pallas import tpu_sc as plsc

NC = 2
NS = 16
NW = NC * NS

NROW = 4096
SEQ = 50
D = 64
RB = NROW // NW
L = 16


def _emb_body(x_hbm, table_hbm, out_hbm, idx_v, list_v, g_v, t_v, drain_v,
              gsem0, gsem1, wsem0, wsem1):
    wid = lax.axis_index("s") * NC + lax.axis_index("c")

    pltpu.sync_copy(x_hbm.at[pl.ds(wid * RB, RB)], idx_v)

    gsems = (gsem0, gsem1)
    wsems = (wsem0, wsem1)

    def build_list(s, b):
        for k in range(RB // L):
            rows = lax.iota(jnp.int32, L) + k * L
            vals = plsc.load_gather(idx_v, [rows, jnp.full((L,), s, jnp.int32)])
            list_v[b, pl.ds(k * L, L)] = vals

    def fire_gather(b):
        pltpu.async_copy(table_hbm.at[list_v.at[b]], g_v.at[b], gsems[b])

    dbase = lax.iota(jnp.int32, L)
    djs = [dbase + j * L for j in range(D // L)]

    def do_s(s, b):
        @pl.when(s >= 2)
        def _():
            pltpu.make_async_copy(
                out_hbm.at[s - 2, :, wid], drain_v, wsems[b]
            ).wait()

        pltpu.make_async_copy(
            table_hbm.at[list_v.at[b]], g_v.at[b], gsems[b]
        ).wait()

        def tr_row(i0):
            for u in range(4):
                i = i0 + u
                col = jnp.full((L,), i, jnp.int32)
                for j in range(D // L):
                    vals = g_v[b, i, pl.ds(j * L, L)]
                    plsc.store_scatter(t_v.at[b], [djs[j], col], vals)

        pl.loop(0, RB, step=4)(tr_row)

        for dt in range(D // 8):
            pltpu.async_copy(
                t_v.at[b, pl.ds(dt * 8, 8), pl.ds(0, RB)],
                out_hbm.at[s, dt, wid],
                wsems[b],
            )

        @pl.when(s + 2 < SEQ)
        def _():
            build_list(s + 2, b)
            fire_gather(b)

    for b in range(2):
        build_list(b, b)
        fire_gather(b)

    def grp(g):
        do_s(g, 0)
        do_s(g + 1, 1)

    pl.loop(0, SEQ, step=2)(grp)

    for b in range(2):
        s = SEQ - 2 + b
        pltpu.make_async_copy(
            out_hbm.at[s, :, wid], drain_v, wsems[b]
        ).wait()


@jax.jit
def kernel(x, table):
    mesh = plsc.VectorSubcoreMesh(core_axis_name="c", subcore_axis_name="s")
    tmp = pl.kernel(
        _emb_body,
        out_type=jax.ShapeDtypeStruct((SEQ, D // 8, NW, 8, RB), jnp.float32),
        mesh=mesh,
        scratch_types=[
            pltpu.VMEM((RB, SEQ), jnp.int32),
            pltpu.VMEM((2, RB), jnp.int32),
            pltpu.VMEM((2, RB, D), jnp.float32),
            pltpu.VMEM((2, D, 129), jnp.float32),
            pltpu.VMEM((D // 8, 8, RB), jnp.float32),
            pltpu.SemaphoreType.DMA,
            pltpu.SemaphoreType.DMA,
            pltpu.SemaphoreType.DMA,
            pltpu.SemaphoreType.DMA,
        ],
        compiler_params=pltpu.CompilerParams(
            use_tc_tiling_on_sc=False, needs_layout_passes=False
        ),
    )(x, table)
    return tmp.transpose(2, 4, 0, 1, 3).reshape(NROW, SEQ, D)


# speedup vs baseline: 1.0522x; 1.0012x over previous
import jax
import jax.numpy as jnp
from jax import lax
from jax.experimental import pallas as pl
from jax.experimental.pallas import tpu as pltpu
from jax.experimental.pallas import tpu_sc as plsc

NC = 2
NS = 16
NW = NC * NS

NROW = 4096
SEQ = 50
D = 64
RB = NROW // NW
L = 16


def _emb_body(x_hbm, table_hbm, out_hbm, idx_v, idx_t, g_v, t_v, drain_v,
              gsem0, gsem1, wsem0, wsem1):
    wid = lax.axis_index("s") * NC + lax.axis_index("c")

    pltpu.sync_copy(x_hbm.at[pl.ds(wid * RB, RB)], idx_v)

    gsems = (gsem0, gsem1)
    wsems = (wsem0, wsem1)

    sbase = lax.iota(jnp.int32, L)
    smask = sbase < (SEQ - 3 * L)

    def tr_idx(i):
        col = jnp.full((L,), i, jnp.int32)
        for k in range(3):
            vals = idx_v[i, pl.ds(k * L, L)]
            plsc.store_scatter(idx_t, [sbase + k * L, col], vals)
        vals = plsc.load_gather(
            idx_v, [jnp.full((L,), i, jnp.int32), sbase + 3 * L], mask=smask
        )
        plsc.store_scatter(idx_t, [sbase + 3 * L, col], vals, mask=smask)

    pl.loop(0, RB)(tr_idx)

    def fire_gather(s, b):
        pltpu.async_copy(
            table_hbm.at[idx_t.at[s, pl.ds(0, RB)]], g_v.at[b], gsems[b]
        )

    dbase = lax.iota(jnp.int32, L)
    djs = [dbase + j * L for j in range(D // L)]

    def do_s(s, b):
        @pl.when(s >= 2)
        def _():
            pltpu.make_async_copy(
                out_hbm.at[s - 2, :, wid], drain_v, wsems[b]
            ).wait()

        pltpu.make_async_copy(
            table_hbm.at[idx_t.at[s, pl.ds(0, RB)]], g_v.at[b], gsems[b]
        ).wait()

        def tr_row(i0):
            for u in range(4):
                i = i0 + u
                col = jnp.full((L,), i, jnp.int32)
                for j in range(D // L):
                    vals = g_v[b, i, pl.ds(j * L, L)]
                    plsc.store_scatter(t_v.at[b], [djs[j], col], vals)

        pl.loop(0, RB, step=4)(tr_row)

        for dt in range(D // 8):
            pltpu.async_copy(
                t_v.at[b, pl.ds(dt * 8, 8), pl.ds(0, RB)],
                out_hbm.at[s, dt, wid],
                wsems[b],
            )

        @pl.when(s + 2 < SEQ)
        def _():
            fire_gather(s + 2, b)

    for b in range(2):
        fire_gather(b, b)

    def grp(g):
        do_s(g, 0)
        do_s(g + 1, 1)

    pl.loop(0, SEQ, step=2)(grp)

    for b in range(2):
        s = SEQ - 2 + b
        pltpu.make_async_copy(
            out_hbm.at[s, :, wid], drain_v, wsems[b]
        ).wait()


@jax.jit
def kernel(x, table):
    mesh = plsc.VectorSubcoreMesh(core_axis_name="c", subcore_axis_name="s")
    tmp = pl.kernel(
        _emb_body,
        out_type=jax.ShapeDtypeStruct((SEQ, D // 8, NW, 8, RB), jnp.float32),
        mesh=mesh,
        scratch_types=[
            pltpu.VMEM((RB, SEQ), jnp.int32),
            pltpu.VMEM((SEQ, 129), jnp.int32),
            pltpu.VMEM((2, RB, D), jnp.float32),
            pltpu.VMEM((2, D, 129), jnp.float32),
            pltpu.VMEM((D // 8, 8, RB), jnp.float32),
            pltpu.SemaphoreType.DMA,
            pltpu.SemaphoreType.DMA,
            pltpu.SemaphoreType.DMA,
            pltpu.SemaphoreType.DMA,
        ],
        compiler_params=pltpu.CompilerParams(
            use_tc_tiling_on_sc=False, needs_layout_passes=False
        ),
    )(x, table)
    return tmp.transpose(2, 4, 0, 1, 3).reshape(NROW, SEQ, D)
